# baseline (device time: 1516390 ns/iter reference)
import jax
import jax.numpy as jnp
from jax import lax
from jax.experimental import pallas as pl
from jax.experimental.pallas import tpu as pltpu

N_DEV = 4


def _neighbor_barrier(left, right):
    barrier_sem = pltpu.get_barrier_semaphore()
    for nbr in [left, right]:
        pl.semaphore_signal(
            barrier_sem, inc=1,
            device_id=(nbr,), device_id_type=pl.DeviceIdType.MESH,
        )
    pl.semaphore_wait(barrier_sem, 2)


def _rdma(src, dst, send_sem, recv_sem, dev):
    return pltpu.make_async_remote_copy(
        src_ref=src, dst_ref=dst, send_sem=send_sem, recv_sem=recv_sem,
        device_id=(dev,), device_id_type=pl.DeviceIdType.MESH,
    )


_NQ = 4


def _ag_w_body(w_ref, wg_ref, local_sem, send_sems, recv_sems):
    my = lax.axis_index("i")
    left = (my - 1) % N_DEV
    right = (my + 1) % N_DEV
    k, n_per = w_ref.shape
    kq = k // _NQ

    def col(c):
        return pl.ds(c * n_per, n_per)

    def qr(q):
        return pl.ds(q * kq, kq)

    _neighbor_barrier(left, right)

    cp = pltpu.make_async_copy(w_ref, wg_ref.at[:, col(my)], local_sem)
    cp.start()

    hop1 = []
    for q in range(_NQ):
        s = _rdma(w_ref.at[qr(q)], wg_ref.at[qr(q), col(my)],
                  send_sems.at[q], recv_sems.at[q], right)
        s.start()
        hop1.append(s)
    for q in range(_NQ):
        s = _rdma(w_ref.at[qr(q)], wg_ref.at[qr(q), col(my)],
                  send_sems.at[_NQ + q], recv_sems.at[_NQ + q], left)
        s.start()
        hop1.append(s)

    hop2 = []
    for q in range(_NQ):
        r = _rdma(wg_ref.at[qr(q), col(left)], wg_ref.at[qr(q), col(left)],
                  send_sems.at[q], recv_sems.at[q], left)
        r.wait_recv()
    for q in range(2):
        s = _rdma(wg_ref.at[qr(q), col(left)], wg_ref.at[qr(q), col(left)],
                  send_sems.at[8 + q], recv_sems.at[8 + q], right)
        s.start()
        hop2.append(s)

    for q in range(_NQ):
        r = _rdma(wg_ref.at[qr(q), col(right)], wg_ref.at[qr(q), col(right)],
                  send_sems.at[_NQ + q], recv_sems.at[_NQ + q], right)
        r.wait_recv()
    for q in range(2, _NQ):
        s = _rdma(wg_ref.at[qr(q), col(right)], wg_ref.at[qr(q), col(right)],
                  send_sems.at[8 + q], recv_sems.at[8 + q], left)
        s.start()
        hop2.append(s)

    opp = (my + 2) % N_DEV
    for q in range(2):
        r = _rdma(wg_ref.at[qr(q), col(opp)], wg_ref.at[qr(q), col(opp)],
                  send_sems.at[8 + q], recv_sems.at[8 + q], left)
        r.wait_recv()
    for q in range(2, _NQ):
        r = _rdma(wg_ref.at[qr(q), col(opp)], wg_ref.at[qr(q), col(opp)],
                  send_sems.at[8 + q], recv_sems.at[8 + q], right)
        r.wait_recv()

    for s in hop1 + hop2:
        s.wait_send()
    cp.wait()


def _ag_w(w_shard):
    k, n_per = w_shard.shape
    return pl.pallas_call(
        _ag_w_body,
        out_shape=jax.ShapeDtypeStruct((k, N_DEV * n_per), w_shard.dtype),
        in_specs=[pl.BlockSpec(memory_space=pl.ANY)],
        out_specs=pl.BlockSpec(memory_space=pl.ANY),
        scratch_shapes=[
            pltpu.SemaphoreType.DMA,
            pltpu.SemaphoreType.DMA((12,)),
            pltpu.SemaphoreType.DMA((12,)),
        ],
        compiler_params=pltpu.CompilerParams(collective_id=0),
    )(w_shard)


def _a2a_body(y_ref, out_ref, transit_ref, local_sem, send_sems, recv_sems):
    my = lax.axis_index("i")
    left = (my - 1) % N_DEV
    right = (my + 1) % N_DEV
    opp = (my + 2) % N_DEV
    m_per, n_full = y_ref.shape
    n_per = n_full // N_DEV

    def col(c):
        return pl.ds(c * n_per, n_per)

    def rows(r):
        return pl.ds(r * m_per, m_per)

    mh = m_per // 2

    def half_rows(r, p):
        return pl.ds(r * m_per + p * mh, mh)

    def half_src(p):
        return pl.ds(p * mh, mh)

    _neighbor_barrier(left, right)

    cp = pltpu.make_async_copy(
        y_ref.at[:, col(my)], out_ref.at[rows(my), :], local_sem
    )
    cp.start()

    sends = []
    for p in range(2):
        s = _rdma(y_ref.at[half_src(p), col(right)],
                  out_ref.at[half_rows(my, p), :],
                  send_sems.at[p], recv_sems.at[p], right)
        s.start()
        sends.append(s)
        s = _rdma(y_ref.at[half_src(p), col(left)],
                  out_ref.at[half_rows(my, p), :],
                  send_sems.at[2 + p], recv_sems.at[2 + p], left)
        s.start()
        sends.append(s)
        s = _rdma(y_ref.at[half_src(p), col(opp)],
                  transit_ref.at[half_src(p)],
                  send_sems.at[4 + p], recv_sems.at[4 + p], right)
        s.start()
        sends.append(s)

    for p in range(2):
        r = _rdma(transit_ref.at[half_src(p)], transit_ref.at[half_src(p)],
                  send_sems.at[4 + p], recv_sems.at[4 + p], left)
        r.wait_recv()
    for p in range(2):
        s = _rdma(transit_ref.at[half_src(p)],
                  out_ref.at[half_rows(left, p), :],
                  send_sems.at[6 + p], recv_sems.at[6 + p], right)
        s.start()
        sends.append(s)

    for p in range(2):
        r = _rdma(out_ref.at[half_rows(left, p), :],
                  out_ref.at[half_rows(left, p), :],
                  send_sems.at[p], recv_sems.at[p], left)
        r.wait_recv()
        r = _rdma(out_ref.at[half_rows(right, p), :],
                  out_ref.at[half_rows(right, p), :],
                  send_sems.at[2 + p], recv_sems.at[2 + p], right)
        r.wait_recv()
        r = _rdma(out_ref.at[half_rows(opp, p), :],
                  out_ref.at[half_rows(opp, p), :],
                  send_sems.at[6 + p], recv_sems.at[6 + p], left)
        r.wait_recv()

    for s in sends:
        s.wait_send()
    cp.wait()


def _a2a(y):
    m_per, n_full = y.shape
    n_per = n_full // N_DEV
    return pl.pallas_call(
        _a2a_body,
        out_shape=jax.ShapeDtypeStruct((N_DEV * m_per, n_per), y.dtype),
        in_specs=[pl.BlockSpec(memory_space=pl.ANY)],
        out_specs=pl.BlockSpec(memory_space=pl.ANY),
        scratch_shapes=[
            pltpu.VMEM((m_per, n_per), y.dtype),
            pltpu.SemaphoreType.DMA,
            pltpu.SemaphoreType.DMA((8,)),
            pltpu.SemaphoreType.DMA((8,)),
        ],
        compiler_params=pltpu.CompilerParams(collective_id=1),
    )(y)


def _gelu(y):
    c = 0.7978845608028654
    return 0.5 * y * (1.0 + jnp.tanh(c * (y + 0.044715 * y * y * y)))


def kernel(x, w_mat):
    w_full = _ag_w(w_mat)
    y = jnp.dot(x, w_full, preferred_element_type=jnp.float32)
    y = _gelu(y).astype(jnp.float32)
    return _a2a(y)
